# baseline (speedup 1.0000x reference)
import functools

import jax
import jax.numpy as jnp
from jax.experimental import pallas as pl
from jax.experimental.pallas import tpu as pltpu

_TILE = 8192


def _gcn_body(adj_ref, x_ref, w_ref, b_ref, o_ref, acc_ref, *, n, tile):
    i = pl.program_id(0)
    nt = pl.num_programs(0)

    @pl.when(i == 0)
    def _init():
        acc_ref[...] = jnp.zeros_like(acc_ref)

    @pl.when(i < nt - 1)
    def _full_tile():
        acc_ref[...] += jnp.dot(
            adj_ref[...], x_ref[...], preferred_element_type=jnp.float32
        )

    @pl.when(i == nt - 1)
    def _tail_tile():
        rem = n - (nt - 1) * tile
        adj_blk = adj_ref[...]
        x_blk = x_ref[...]
        if rem != tile:
            cols = jax.lax.broadcasted_iota(jnp.int32, adj_blk.shape, 1)
            rows = jax.lax.broadcasted_iota(jnp.int32, x_blk.shape, 0)
            adj_blk = jnp.where(cols < rem, adj_blk, 0.0)
            x_blk = jnp.where(rows < rem, x_blk, 0.0)
        acc = acc_ref[...] + jnp.dot(
            adj_blk, x_blk, preferred_element_type=jnp.float32
        )
        o_ref[...] = (
            jnp.dot(acc, w_ref[...], preferred_element_type=jnp.float32)
            + b_ref[...]
        )


@jax.jit
def kernel(x, adj, weight, bias):
    n, f_in = x.shape
    f_out = adj.shape[0]
    tile = min(_TILE, n)
    nt = pl.cdiv(n, tile)
    bias2 = bias.reshape(1, f_out)
    return pl.pallas_call(
        functools.partial(_gcn_body, n=n, tile=tile),
        grid=(nt,),
        in_specs=[
            pl.BlockSpec((f_out, tile), lambda i: (0, i)),
            pl.BlockSpec((tile, f_in), lambda i: (i, 0)),
            pl.BlockSpec((f_in, f_out), lambda i: (0, 0)),
            pl.BlockSpec((1, f_out), lambda i: (0, 0)),
        ],
        out_specs=pl.BlockSpec((f_out, f_out), lambda i: (0, 0)),
        out_shape=jax.ShapeDtypeStruct((f_out, f_out), jnp.float32),
        scratch_shapes=[pltpu.VMEM((f_out, f_out), jnp.float32)],
        compiler_params=pltpu.CompilerParams(
            dimension_semantics=("arbitrary",)
        ),
    )(adj, x, weight, bias2)
